# baseline (speedup 1.0000x reference)
import jax
import jax.numpy as jnp
import numpy as np
from jax.experimental import pallas as pl
from jax.experimental.pallas import tpu as pltpu

_INPUT_SIZE = 608
_FS = 76
_NC = 80
_ANCHORS = np.array([[10.0, 13.0], [16.0, 30.0], [33.0, 23.0]], dtype=np.float32)
_SCALE = _INPUT_SIZE / _FS
_AW = (_ANCHORS[:, 0] / _SCALE).tolist()
_AH = (_ANCHORS[:, 1] / _SCALE).tolist()
_NOOBJ = 0.5
_OBJ = 5.0
_NB = 16
_NGT = 20
_NA = 3
_N = _NB * _NGT
_NSTEP = _NB * _NA
_PER = 8
_LAG = 2 * _PER
_W = 32


def _bce(x, t):
    return jnp.maximum(x, 0.0) - x * t + jnp.log1p(jnp.exp(-jnp.abs(x)))


def _gt_copy(k, b_pf, gj_pf, gi_pf, hbm_ref, g_all, sems):
    return pltpu.make_async_copy(
        hbm_ref.at[gj_pf[k], gi_pf[k], b_pf[k]],
        g_all.at[k],
        sems.at[jax.lax.rem(k, _W)],
    )


def _correction(g_all, bb_ref, bbT_ref, lab_ref, ba_ref, baT_ref, out_ref):
    n = _N
    inv_s = 1.0 / _SCALE

    gx = bb_ref[:, 0:1] * inv_s
    gy = bb_ref[:, 1:2] * inv_s
    gw = bb_ref[:, 2:3] * inv_s
    gh = bb_ref[:, 3:4] * inv_s
    gi = jnp.floor(gx)
    gj = jnp.floor(gy)
    baf = ba_ref[:, 0:1].astype(jnp.float32)
    kcol = jax.lax.broadcasted_iota(jnp.int32, (n, 1), 0).astype(jnp.float32)
    bcol = jnp.floor(kcol * (1.0 / _NGT))

    gxr = bbT_ref[0:1, :] * inv_s
    gyr = bbT_ref[1:2, :] * inv_s
    gwr = bbT_ref[2:3, :] * inv_s
    ghr = bbT_ref[3:4, :] * inv_s
    gir = jnp.floor(gxr)
    gjr = jnp.floor(gyr)
    bafr = baT_ref[0:1, :].astype(jnp.float32)
    krow = jax.lax.broadcasted_iota(jnp.int32, (1, n), 1).astype(jnp.float32)
    brow = jnp.floor(krow * (1.0 / _NGT))

    keyc = ((bcol * _NA + baf) * _FS + gj) * _FS + gi
    keyr = ((brow * _NA + bafr) * _FS + gjr) * _FS + gir
    dup_later = (keyc == keyr) & (krow > kcol)
    alive = 1.0 - jnp.clip(jnp.sum(dup_later.astype(jnp.float32), axis=1,
                                   keepdims=True), 0.0, 1.0)

    cellc = (bcol * _FS + gj) * _FS + gi
    cellr = (brow * _FS + gjr) * _FS + gir
    same = (cellc == cellr).astype(jnp.float32)
    onehot = (lab_ref[:, 0:1] ==
              jax.lax.broadcasted_iota(jnp.int32, (n, _NC), 1)).astype(jnp.float32)
    multi = (jax.lax.dot(same, onehot,
                         preferred_element_type=jnp.float32) > 0.0).astype(jnp.float32)

    rows = g_all[...]
    g = ((baf == 0.0) * rows[:, 0:85]
         + (baf == 1.0) * rows[:, 85:170]
         + (baf == 2.0) * rows[:, 170:255])

    x = g[:, 0:1]
    y = g[:, 1:2]
    w = g[:, 2:3]
    h = g[:, 3:4]
    conf = g[:, 4:5]
    cls = g[:, 5:85]

    aw = jnp.where(baf == 0.0, _AW[0], jnp.where(baf == 1.0, _AW[1], _AW[2]))
    ah = jnp.where(baf == 0.0, _AH[0], jnp.where(baf == 1.0, _AH[1], _AH[2]))
    sx = jax.nn.sigmoid(x)
    sy = jax.nn.sigmoid(y)
    pw = jnp.exp(w) * aw
    ph = jnp.exp(h) * ah
    px1 = (sx + gi) - pw * 0.5
    py1 = (sy + gj) - ph * 0.5
    px2 = px1 + pw
    py2 = py1 + ph
    parea = pw * ph

    ox1 = gxr - gwr * 0.5
    oy1 = gyr - ghr * 0.5
    ox2 = ox1 + gwr
    oy2 = oy1 + ghr
    oarea = gwr * ghr
    iw = jnp.maximum(jnp.minimum(px2, ox2) - jnp.maximum(px1, ox1), 0.0)
    ih = jnp.maximum(jnp.minimum(py2, oy2) - jnp.maximum(py1, oy1), 0.0)
    inter = iw * ih
    over = (brow == bcol) & (3.0 * inter > parea + oarea)
    ignore = jnp.sum(over.astype(jnp.float32), axis=1, keepdims=True) > 0.0
    w0 = jnp.where(ignore, 0.0, _NOOBJ)

    tx = gx - gi
    ty = gy - gj
    tw = jnp.log(gw / aw)
    th = jnp.log(gh / ah)

    tgt_coord = (sx - tx) ** 2 + (sy - ty) ** 2 + (w - tw) ** 2 + (h - th) ** 2
    dense_coord = (sx - 0.5) ** 2 + (sy - 0.5) ** 2 + w * w + h * h
    sp_pos = jnp.maximum(conf, 0.0) + jnp.log1p(jnp.exp(-jnp.abs(conf)))
    conf_corr = _OBJ * (sp_pos - conf) - w0 * sp_pos
    cls_sum = jnp.sum(_bce(cls, multi), axis=1, keepdims=True)

    out_ref[0, 0] = jnp.sum(alive * tgt_coord)
    out_ref[0, 1] = jnp.sum(alive * dense_coord)
    out_ref[0, 2] = jnp.sum(alive * conf_corr)
    out_ref[0, 3] = jnp.sum(alive * cls_sum)


def _fused_body(b_pf, ba_pf, gj_pf, gi_pf, data_ref, bbs_ref, hbm_ref,
                bb_ref, bbT_ref, lab_ref, ba_ref, baT_ref,
                s_conf_ref, s_coord_ref, out_ref, g_all, sems):
    b = pl.program_id(0)
    a = pl.program_id(1)
    s = b * _NA + a

    for t in range(_PER):
        k = s * _PER + t
        kprev = k - _LAG

        @pl.when((kprev >= 0) & (kprev < _N))
        def _():
            _gt_copy(kprev, b_pf, gj_pf, gi_pf, hbm_ref, g_all, sems).wait()

        @pl.when(k < _N)
        def _():
            _gt_copy(k, b_pf, gj_pf, gi_pf, hbm_ref, g_all, sems).start()

    aw = jnp.where(a == 0, _AW[0], jnp.where(a == 1, _AW[1], _AW[2]))
    ah = jnp.where(a == 0, _AH[0], jnp.where(a == 1, _AH[1], _AH[2]))

    x = data_ref[0, 0, 0]
    y = data_ref[0, 0, 1]
    w = data_ref[0, 0, 2]
    h = data_ref[0, 0, 3]
    conf = data_ref[0, 0, 4]

    cellx = jax.lax.broadcasted_iota(jnp.int32, (_FS, _FS), 1).astype(jnp.float32)
    celly = jax.lax.broadcasted_iota(jnp.int32, (_FS, _FS), 0).astype(jnp.float32)

    sx = jax.nn.sigmoid(x)
    sy = jax.nn.sigmoid(y)
    pw = jnp.exp(w) * aw
    ph = jnp.exp(h) * ah
    px1 = (sx + cellx) - pw * 0.5
    py1 = (sy + celly) - ph * 0.5
    px2 = px1 + pw
    py2 = py1 + ph
    pa3 = pw * ph * (1.0 / 3.0)

    ignore = jnp.zeros((_FS, _FS), dtype=jnp.bool_)
    inv_s = 1.0 / _SCALE
    for g in range(_NGT):
        gx = bbs_ref[b, g, 0] * inv_s
        gy = bbs_ref[b, g, 1] * inv_s
        gw = bbs_ref[b, g, 2] * inv_s
        gh = bbs_ref[b, g, 3] * inv_s
        gx1 = gx - gw * 0.5
        gy1 = gy - gh * 0.5
        gx2 = gx1 + gw
        gy2 = gy1 + gh
        ga3 = gw * gh * (1.0 / 3.0)
        iw = jnp.maximum(jnp.minimum(px2, gx2) - jnp.maximum(px1, gx1), 0.0)
        ih = jnp.maximum(jnp.minimum(py2, gy2) - jnp.maximum(py1, gy1), 0.0)
        ignore = ignore | (iw * ih > pa3 + ga3)

    weight = jnp.where(ignore, 0.0, _NOOBJ)
    sp = jnp.maximum(conf, 0.0) + jnp.log1p(jnp.exp(-jnp.abs(conf)))
    s_conf_ref[b, a] = jnp.sum(weight * sp)
    s_coord_ref[b, a] = jnp.sum((sx - 0.5) ** 2 + (sy - 0.5) ** 2 + w * w + h * h)

    @pl.when(s == _NSTEP - 1)
    def _():
        _correction(g_all, bb_ref, bbT_ref, lab_ref, ba_ref, baT_ref, out_ref)


def kernel(output, bboxes_group, labels_group, best_anchors_idx_group, warm_up):
    nB, nA, nGT = _NB, _NA, _NGT
    n = _N
    wu = jnp.asarray(warm_up).astype(jnp.float32)

    chs = output.reshape(nB, nA, 85, _FS, _FS)[:, :, 0:5]
    outT = jnp.transpose(output, (2, 3, 0, 1))

    gt_ij = (bboxes_group[..., :2] * (1.0 / _SCALE)).astype(jnp.int32)
    gi_s = gt_ij[..., 0].reshape(-1)
    gj_s = gt_ij[..., 1].reshape(-1)
    ba_s = best_anchors_idx_group.astype(jnp.int32).reshape(-1)
    b_s = jax.lax.broadcasted_iota(jnp.int32, (nB, nGT), 0).reshape(-1)

    bb2 = bboxes_group.reshape(n, 4)
    bbT = jnp.transpose(bb2)
    lab2 = labels_group.astype(jnp.int32).reshape(n, 1)
    ba2 = ba_s.reshape(n, 1)
    baT = ba_s.reshape(1, n)

    s_conf, s_coord, corr = pl.pallas_call(
        _fused_body,
        grid_spec=pltpu.PrefetchScalarGridSpec(
            num_scalar_prefetch=4,
            grid=(nB, nA),
            in_specs=[
                pl.BlockSpec((1, 1, 5, _FS, _FS),
                             lambda b, a, *_: (b, a, 0, 0, 0)),
                pl.BlockSpec(memory_space=pltpu.SMEM),
                pl.BlockSpec(memory_space=pltpu.MemorySpace.HBM),
                pl.BlockSpec(memory_space=pltpu.VMEM),
                pl.BlockSpec(memory_space=pltpu.VMEM),
                pl.BlockSpec(memory_space=pltpu.VMEM),
                pl.BlockSpec(memory_space=pltpu.VMEM),
                pl.BlockSpec(memory_space=pltpu.VMEM),
            ],
            out_specs=[
                pl.BlockSpec(memory_space=pltpu.SMEM),
                pl.BlockSpec(memory_space=pltpu.SMEM),
                pl.BlockSpec(memory_space=pltpu.SMEM),
            ],
            scratch_shapes=[
                pltpu.VMEM((n, 255), jnp.float32),
                pltpu.SemaphoreType.DMA((_W,)),
            ],
        ),
        out_shape=[
            jax.ShapeDtypeStruct((nB, nA), jnp.float32),
            jax.ShapeDtypeStruct((nB, nA), jnp.float32),
            jax.ShapeDtypeStruct((1, 4), jnp.float32),
        ],
    )(b_s, ba_s, gj_s, gi_s,
      chs, bboxes_group, outT, bb2, bbT, lab2, ba2, baT)

    loss_coord = (wu * (jnp.sum(s_coord) - corr[0, 1]) + corr[0, 0]) / 2.0 / nB
    loss_conf = (jnp.sum(s_conf) + corr[0, 2]) / nB
    loss_cls = corr[0, 3] / nB
    return loss_coord + loss_conf + loss_cls


# speedup vs baseline: 2.7644x; 2.7644x over previous
import jax
import jax.numpy as jnp
import numpy as np
from jax.experimental import pallas as pl
from jax.experimental.pallas import tpu as pltpu

_INPUT_SIZE = 608
_FS = 76
_NC = 80
_ANCHORS = np.array([[10.0, 13.0], [16.0, 30.0], [33.0, 23.0]], dtype=np.float32)
_SCALE = _INPUT_SIZE / _FS
_AW = (_ANCHORS[:, 0] / _SCALE).tolist()
_AH = (_ANCHORS[:, 1] / _SCALE).tolist()
_NOOBJ = 0.5
_OBJ = 5.0
_NB = 16
_NGT = 20
_NA = 3
_N = _NB * _NGT
_NSTEP = _NB * _NA
_PER = 8
_RING = 2 * _PER
_W = 32


def _bce(x, t):
    return jnp.maximum(x, 0.0) - x * t + jnp.log1p(jnp.exp(-jnp.abs(x)))


def _gt_copy(k, slot, b_pf, ba_pf, gj_pf, hbm_ref, ring, sems):
    return pltpu.make_async_copy(
        hbm_ref.at[b_pf[k], pl.ds(ba_pf[k] * 85, 85), gj_pf[k]],
        ring.at[slot],
        sems.at[jax.lax.rem(k, _W)],
    )


def _correction(g_scr, bb_ref, bbT_ref, lab_ref, ba_ref, baT_ref, out_ref):
    n = _N
    inv_s = 1.0 / _SCALE

    gx = bb_ref[:, 0:1] * inv_s
    gy = bb_ref[:, 1:2] * inv_s
    gw = bb_ref[:, 2:3] * inv_s
    gh = bb_ref[:, 3:4] * inv_s
    gi = jnp.floor(gx)
    gj = jnp.floor(gy)
    baf = ba_ref[:, 0:1].astype(jnp.float32)
    kcol = jax.lax.broadcasted_iota(jnp.int32, (n, 1), 0).astype(jnp.float32)
    bcol = jnp.floor(kcol * (1.0 / _NGT))

    gxr = bbT_ref[0:1, :] * inv_s
    gyr = bbT_ref[1:2, :] * inv_s
    gwr = bbT_ref[2:3, :] * inv_s
    ghr = bbT_ref[3:4, :] * inv_s
    gir = jnp.floor(gxr)
    gjr = jnp.floor(gyr)
    bafr = baT_ref[0:1, :].astype(jnp.float32)
    krow = jax.lax.broadcasted_iota(jnp.int32, (1, n), 1).astype(jnp.float32)
    brow = jnp.floor(krow * (1.0 / _NGT))

    keyc = ((bcol * _NA + baf) * _FS + gj) * _FS + gi
    keyr = ((brow * _NA + bafr) * _FS + gjr) * _FS + gir
    dup_later = (keyc == keyr) & (krow > kcol)
    alive = 1.0 - jnp.clip(jnp.sum(dup_later.astype(jnp.float32), axis=1,
                                   keepdims=True), 0.0, 1.0)

    cellc = (bcol * _FS + gj) * _FS + gi
    cellr = (brow * _FS + gjr) * _FS + gir
    same = (cellc == cellr).astype(jnp.float32)
    onehot = (lab_ref[:, 0:1] ==
              jax.lax.broadcasted_iota(jnp.int32, (n, _NC), 1)).astype(jnp.float32)
    multi = (jax.lax.dot(same, onehot,
                         preferred_element_type=jnp.float32) > 0.0).astype(jnp.float32)

    g = g_scr[...]
    x = g[:, 0:1]
    y = g[:, 1:2]
    w = g[:, 2:3]
    h = g[:, 3:4]
    conf = g[:, 4:5]
    cls = g[:, 5:85]

    aw = jnp.where(baf == 0.0, _AW[0], jnp.where(baf == 1.0, _AW[1], _AW[2]))
    ah = jnp.where(baf == 0.0, _AH[0], jnp.where(baf == 1.0, _AH[1], _AH[2]))
    sx = jax.nn.sigmoid(x)
    sy = jax.nn.sigmoid(y)
    pw = jnp.exp(w) * aw
    ph = jnp.exp(h) * ah
    px1 = (sx + gi) - pw * 0.5
    py1 = (sy + gj) - ph * 0.5
    px2 = px1 + pw
    py2 = py1 + ph
    parea = pw * ph

    ox1 = gxr - gwr * 0.5
    oy1 = gyr - ghr * 0.5
    ox2 = ox1 + gwr
    oy2 = oy1 + ghr
    oarea = gwr * ghr
    iw = jnp.maximum(jnp.minimum(px2, ox2) - jnp.maximum(px1, ox1), 0.0)
    ih = jnp.maximum(jnp.minimum(py2, oy2) - jnp.maximum(py1, oy1), 0.0)
    inter = iw * ih
    over = (brow == bcol) & (3.0 * inter > parea + oarea)
    ignore = jnp.sum(over.astype(jnp.float32), axis=1, keepdims=True) > 0.0
    w0 = jnp.where(ignore, 0.0, _NOOBJ)

    tx = gx - gi
    ty = gy - gj
    tw = jnp.log(gw / aw)
    th = jnp.log(gh / ah)

    tgt_coord = (sx - tx) ** 2 + (sy - ty) ** 2 + (w - tw) ** 2 + (h - th) ** 2
    dense_coord = (sx - 0.5) ** 2 + (sy - 0.5) ** 2 + w * w + h * h
    sp_pos = jnp.maximum(conf, 0.0) + jnp.log1p(jnp.exp(-jnp.abs(conf)))
    conf_corr = _OBJ * (sp_pos - conf) - w0 * sp_pos
    cls_sum = jnp.sum(_bce(cls, multi), axis=1, keepdims=True)

    out_ref[0, 0] = jnp.sum(alive * tgt_coord)
    out_ref[0, 1] = jnp.sum(alive * dense_coord)
    out_ref[0, 2] = jnp.sum(alive * conf_corr)
    out_ref[0, 3] = jnp.sum(alive * cls_sum)


def _fused_body(b_pf, ba_pf, gj_pf, gi_pf, data_ref, bbs_ref, hbm_ref,
                bb_ref, bbT_ref, lab_ref, ba_ref, baT_ref, gif_ref,
                s_conf_ref, s_coord_ref, out_ref, ring, g_cmp, sems):
    b = pl.program_id(0)
    a = pl.program_id(1)
    s = b * _NA + a
    rbase = 8 * jax.lax.rem(s, 2)

    kp0 = (s - 2) * _PER

    @pl.when((s >= 2) & (kp0 < _N))
    def _():
        for t in range(_PER):
            _gt_copy(kp0 + t, rbase + t, b_pf, ba_pf, gj_pf,
                     hbm_ref, ring, sems).wait()
        rows = ring[pl.ds(rbase, _PER)]
        giv = gif_ref[pl.ds(pl.multiple_of(kp0, 8), _PER)]
        lane = jax.lax.broadcasted_iota(
            jnp.int32, (_PER, 1, _FS), 2).astype(jnp.float32)
        oh = (lane == giv[:, :, None]).astype(jnp.float32)
        vals = jnp.sum(rows * oh, axis=2)
        g_cmp[pl.ds(pl.multiple_of(kp0, 8), _PER), :] = vals

    for t in range(_PER):
        k = s * _PER + t

        @pl.when(k < _N)
        def _():
            _gt_copy(k, rbase + t, b_pf, ba_pf, gj_pf,
                     hbm_ref, ring, sems).start()

    aw = jnp.where(a == 0, _AW[0], jnp.where(a == 1, _AW[1], _AW[2]))
    ah = jnp.where(a == 0, _AH[0], jnp.where(a == 1, _AH[1], _AH[2]))

    x = data_ref[0, 0]
    y = data_ref[0, 1]
    w = data_ref[0, 2]
    h = data_ref[0, 3]
    conf = data_ref[0, 4]

    cellx = jax.lax.broadcasted_iota(jnp.int32, (_FS, _FS), 1).astype(jnp.float32)
    celly = jax.lax.broadcasted_iota(jnp.int32, (_FS, _FS), 0).astype(jnp.float32)

    sx = jax.nn.sigmoid(x)
    sy = jax.nn.sigmoid(y)
    pw = jnp.exp(w) * aw
    ph = jnp.exp(h) * ah
    px1 = (sx + cellx) - pw * 0.5
    py1 = (sy + celly) - ph * 0.5
    px2 = px1 + pw
    py2 = py1 + ph
    pa3 = pw * ph * (1.0 / 3.0)

    ignore = jnp.zeros((_FS, _FS), dtype=jnp.bool_)
    inv_s = 1.0 / _SCALE
    for g in range(_NGT):
        gx = bbs_ref[b, g, 0] * inv_s
        gy = bbs_ref[b, g, 1] * inv_s
        gw = bbs_ref[b, g, 2] * inv_s
        gh = bbs_ref[b, g, 3] * inv_s
        gx1 = gx - gw * 0.5
        gy1 = gy - gh * 0.5
        gx2 = gx1 + gw
        gy2 = gy1 + gh
        ga3 = gw * gh * (1.0 / 3.0)
        iw = jnp.maximum(jnp.minimum(px2, gx2) - jnp.maximum(px1, gx1), 0.0)
        ih = jnp.maximum(jnp.minimum(py2, gy2) - jnp.maximum(py1, gy1), 0.0)
        ignore = ignore | (iw * ih > pa3 + ga3)

    weight = jnp.where(ignore, 0.0, _NOOBJ)
    sp = jnp.maximum(conf, 0.0) + jnp.log1p(jnp.exp(-jnp.abs(conf)))
    s_conf_ref[b, a] = jnp.sum(weight * sp)
    s_coord_ref[b, a] = jnp.sum((sx - 0.5) ** 2 + (sy - 0.5) ** 2 + w * w + h * h)

    @pl.when(s == _NSTEP - 1)
    def _():
        _correction(g_cmp, bb_ref, bbT_ref, lab_ref, ba_ref, baT_ref, out_ref)


def kernel(output, bboxes_group, labels_group, best_anchors_idx_group, warm_up):
    nB, nA, nGT = _NB, _NA, _NGT
    n = _N
    wu = jnp.asarray(warm_up).astype(jnp.float32)

    gt_ij = (bboxes_group[..., :2] * (1.0 / _SCALE)).astype(jnp.int32)
    gi_s = gt_ij[..., 0].reshape(-1)
    gj_s = gt_ij[..., 1].reshape(-1)
    ba_s = best_anchors_idx_group.astype(jnp.int32).reshape(-1)
    b_s = jax.lax.broadcasted_iota(jnp.int32, (nB, nGT), 0).reshape(-1)

    bb2 = bboxes_group.reshape(n, 4)
    bbT = jnp.transpose(bb2)
    lab2 = labels_group.astype(jnp.int32).reshape(n, 1)
    ba2 = ba_s.reshape(n, 1)
    baT = ba_s.reshape(1, n)

    s_conf, s_coord, corr = pl.pallas_call(
        _fused_body,
        grid_spec=pltpu.PrefetchScalarGridSpec(
            num_scalar_prefetch=4,
            grid=(nB, nA),
            in_specs=[
                pl.BlockSpec((1, 5, _FS, _FS), lambda b, a, *_: (b, 17 * a, 0, 0)),
                pl.BlockSpec(memory_space=pltpu.SMEM),
                pl.BlockSpec(memory_space=pltpu.MemorySpace.HBM),
                pl.BlockSpec(memory_space=pltpu.VMEM),
                pl.BlockSpec(memory_space=pltpu.VMEM),
                pl.BlockSpec(memory_space=pltpu.VMEM),
                pl.BlockSpec(memory_space=pltpu.VMEM),
                pl.BlockSpec(memory_space=pltpu.VMEM),
                pl.BlockSpec(memory_space=pltpu.VMEM),
            ],
            out_specs=[
                pl.BlockSpec(memory_space=pltpu.SMEM),
                pl.BlockSpec(memory_space=pltpu.SMEM),
                pl.BlockSpec(memory_space=pltpu.SMEM),
            ],
            scratch_shapes=[
                pltpu.VMEM((_RING, 85, _FS), jnp.float32),
                pltpu.VMEM((n, 85), jnp.float32),
                pltpu.SemaphoreType.DMA((_W,)),
            ],
        ),
        out_shape=[
            jax.ShapeDtypeStruct((nB, nA), jnp.float32),
            jax.ShapeDtypeStruct((nB, nA), jnp.float32),
            jax.ShapeDtypeStruct((1, 4), jnp.float32),
        ],
    )(b_s, ba_s, gj_s, gi_s,
      output, bboxes_group, output, bb2, bbT, lab2, ba2, baT,
      gi_s.astype(jnp.float32).reshape(n, 1))

    loss_coord = (wu * (jnp.sum(s_coord) - corr[0, 1]) + corr[0, 0]) / 2.0 / nB
    loss_conf = (jnp.sum(s_conf) + corr[0, 2]) / nB
    loss_cls = corr[0, 3] / nB
    return loss_coord + loss_conf + loss_cls
